# baseline (speedup 1.0000x reference)
import jax
import jax.numpy as jnp
from jax import lax
from jax.experimental import pallas as pl
from jax.experimental.pallas import tpu as pltpu
from jax.experimental.pallas import tpu_sc as plsc

HIDDEN = 1024
INTERMEDIATE = 8192
E = 64
VOCAB = 100000
EI = INTERMEDIATE // E
N = 8192

T = 128
NT = N // T
MAX_STEPS = NT + E

NC = 2
NS = 16
NW = NC * NS
L = 16
CHUNK = N // NW
VREGS = CHUNK // L
ROWS = 64
SUB = CHUNK // ROWS
MROWS = 10

def _sc_mesh():
    return plsc.VectorSubcoreMesh(core_axis_name="c", subcore_axis_name="s",
                                  num_cores=NC, num_subcores=NS)


def _wid():
    return lax.axis_index("s") * NC + lax.axis_index("c")


def _run_ranks(sk, scratch_ref):
    lane = lax.iota(jnp.int32, L)
    scratch_ref[...] = sk
    prev = plsc.load_gather(scratch_ref, [jnp.maximum(lane - 1, 0)])
    nxt = plsc.load_gather(scratch_ref, [jnp.minimum(lane + 1, L - 1)])
    is_start = (lane == 0) | (sk != prev)
    is_last = (lane == L - 1) | (sk != nxt)
    run_start = plsc.cummax(jnp.where(is_start, lane, 0))
    return lane - run_start, is_last


def _route_body(tids_hbm, tte_hbm, e_hbm, lr_hbm, hist_hbm,
                ids_v, tbl_v, e_v, lr_v, hist_v, tmp_v):
    wid = _wid()
    pltpu.sync_copy(tids_hbm.at[pl.ds(wid * CHUNK, CHUNK)], ids_v)
    pltpu.sync_copy(tte_hbm, tbl_v)
    lane = lax.iota(jnp.int32, L)
    for g in range(E // L):
        hist_v[pl.ds(g * L, L)] = jnp.zeros((L,), jnp.int32)
    for v in range(VREGS):
        idv = ids_v[pl.ds(v * L, L)]
        idv = jnp.minimum(jnp.maximum(idv, 0), VOCAB - 1)
        e16 = plsc.load_gather(tbl_v, [idv])
        e_v[pl.ds(v * L, L)] = e16
        sk, sv = plsc.sort_key_val(e16, lane)
        r, is_last = _run_ranks(sk, tmp_v)
        base = plsc.load_gather(hist_v, [sk])
        plsc.store_scatter(tmp_v, [sv], base + r)
        lr_v[pl.ds(v * L, L)] = tmp_v[...]
        plsc.addupdate_scatter(hist_v, [sk], r + 1, mask=is_last)
    pltpu.sync_copy(e_v, e_hbm.at[wid])
    pltpu.sync_copy(lr_v, lr_hbm.at[wid])
    pltpu.sync_copy(hist_v, hist_hbm.at[wid])


def _sc_route(token_ids, token_to_expert):
    f = pl.kernel(
        _route_body,
        out_type=(
            jax.ShapeDtypeStruct((NW, CHUNK), jnp.int32),
            jax.ShapeDtypeStruct((NW, CHUNK), jnp.int32),
            jax.ShapeDtypeStruct((NW, E), jnp.int32),
        ),
        mesh=_sc_mesh(),
        compiler_params=pltpu.CompilerParams(needs_layout_passes=False),
        scratch_types=[
            pltpu.VMEM((CHUNK,), jnp.int32),
            pltpu.VMEM((VOCAB,), jnp.int32),
            pltpu.VMEM((CHUNK,), jnp.int32),
            pltpu.VMEM((CHUNK,), jnp.int32),
            pltpu.VMEM((E,), jnp.int32),
            pltpu.VMEM((L,), jnp.int32),
        ],
    )
    return f(token_ids, token_to_expert)


_NEG = -(2 ** 30)


def _cummax_fill(buf_ref):
    carry = jnp.int32(_NEG)
    for q in range(MAX_STEPS // L):
        v = plsc.cummax(jnp.maximum(buf_ref[pl.ds(q * L, L)], carry))
        buf_ref[pl.ds(q * L, L)] = v
        carry = jnp.max(v)


def _build_meta(st_v, en_v, tot_by_g, total_tokens_unused, tb_v, eb_v, meta_v):
    lane = lax.iota(jnp.int32, L)
    carry = jnp.int32(0)
    for q in range(MAX_STEPS // L):
        tb_v[pl.ds(q * L, L)] = jnp.full((L,), _NEG, jnp.int32)
        eb_v[pl.ds(q * L, L)] = jnp.full((L,), _NEG, jnp.int32)
    for g in range(E // L):
        ev = jnp.int32(g * L) + lane
        starts = st_v[pl.ds(g * L, L)]
        ends = en_v[pl.ds(g * L, L)]
        tot = tot_by_g[g]
        ft = starts // T
        lt = jnp.maximum(ends - 1, 0) // T
        nsteps = jnp.where(tot > 0, lt - ft + 1, 0)
        inc = plsc.cumsum(nsteps) + carry
        off = inc - nsteps
        carry = jnp.max(inc)
        nz = tot > 0
        plsc.store_scatter(tb_v, [jnp.where(nz, off, 0)], off - ft, mask=nz)
        plsc.store_scatter(eb_v, [jnp.where(nz, off, 0)], ev, mask=nz)
    total = carry
    _cummax_fill(tb_v)
    _cummax_fill(eb_v)
    for q in range(MAX_STEPS // L):
        w = jnp.int32(q * L) + lane
        tile = jnp.minimum(w - tb_v[pl.ds(q * L, L)], NT - 1)
        e_w = eb_v[pl.ds(q * L, L)]
        starts = plsc.load_gather(st_v, [e_w])
        ends = plsc.load_gather(en_v, [e_w])
        valid = w < total
        rs = jnp.where(valid, jnp.maximum(starts - tile * T, 0), 0)
        re_ = jnp.where(valid, jnp.minimum(ends - tile * T, T), 0)
        meta_v[0, pl.ds(q * L, L)] = tile
        meta_v[1, pl.ds(q * L, L)] = e_w
        meta_v[2, pl.ds(q * L, L)] = rs
        meta_v[3, pl.ds(q * L, L)] = re_
    for q in range(MAX_STEPS // L):
        w = jnp.int32(q * L) + lane
        tile = meta_v[0, pl.ds(q * L, L)]
        prev = plsc.load_gather(tb_v, [jnp.maximum(w - 1, 0)])
        prev = jnp.minimum(jnp.maximum(w - 1, 0) - prev, NT - 1)
        meta_v[4, pl.ds(q * L, L)] = ((w == 0) | (tile != prev)).astype(jnp.int32)
    fcarry = jnp.int32(0)
    wcarry = jnp.int32(0)
    one_v = jnp.full((L,), 1, jnp.int32)
    for q in range(MAX_STEPS // L):
        w = jnp.int32(q * L) + lane
        e_w = meta_v[1, pl.ds(q * L, L)]
        prev_e = plsc.load_gather(eb_v, [jnp.maximum(w - 1, 0)])
        wchg = ((w == 0) | (e_w != prev_e)).astype(jnp.int32)
        meta_v[5, pl.ds(q * L, L)] = wchg
        first = meta_v[4, pl.ds(q * L, L)]
        visits = plsc.cumsum(first) + fcarry
        fcarry = jnp.max(visits)
        wcnt = plsc.cumsum(wchg) + wcarry
        wcarry = jnp.max(wcnt)
        meta_v[6, pl.ds(q * L, L)] = (visits - 1) & one_v
        meta_v[7, pl.ds(q * L, L)] = (wcnt - 1) & one_v
        meta_v[8, pl.ds(q * L, L)] = first * (visits >= 3).astype(jnp.int32)
    for q in range(MAX_STEPS // L):
        w = jnp.int32(q * L) + lane
        nxt_first = plsc.load_gather(
            meta_v, [jnp.full((L,), 4, jnp.int32),
                     jnp.minimum(w + 1, MAX_STEPS - 1)])
        meta_v[9, pl.ds(q * L, L)] = jnp.where(
            w == MAX_STEPS - 1, 1, nxt_first)


def _scatter_body(x_hbm, e_hbm, lr_hbm, hist_hbm, sx_hbm, dest_hbm, meta_hbm,
                  hv, base_v, e_v, lr_v, dest_v, xbuf, st_v, en_v, tb_v, eb_v,
                  meta_v, sem):
    wid = _wid()
    pltpu.sync_copy(hist_hbm, hv)
    pltpu.sync_copy(e_hbm.at[wid], e_v)
    pltpu.sync_copy(lr_hbm.at[wid], lr_v)
    carry = jnp.int32(0)
    tot_by_g = []
    for g in range(E // L):
        tot = jnp.zeros((L,), jnp.int32)
        mine = jnp.zeros((L,), jnp.int32)
        for t in range(NW):
            h = hv[t, pl.ds(g * L, L)]
            tot = tot + h
            mine = mine + h * (jnp.int32(t) < wid).astype(jnp.int32)
        excl = plsc.cumsum(tot) - tot
        starts = excl + carry
        st_v[pl.ds(g * L, L)] = starts
        en_v[pl.ds(g * L, L)] = starts + tot
        base_v[pl.ds(g * L, L)] = starts + mine
        carry = carry + jnp.sum(tot)
        tot_by_g.append(tot)

    @pl.when(wid == 0)
    def _():
        _build_meta(st_v, en_v, tot_by_g, carry, tb_v, eb_v, meta_v)
        pltpu.sync_copy(meta_v, meta_hbm)

    for v in range(VREGS):
        e16 = e_v[pl.ds(v * L, L)]
        lr16 = lr_v[pl.ds(v * L, L)]
        d16 = plsc.load_gather(base_v, [e16]) + lr16
        dest_v[v // (ROWS // L), pl.ds((v % (ROWS // L)) * L, L)] = d16
    pltpu.sync_copy(dest_v, dest_hbm.at[wid])
    for k in range(SUB):
        pltpu.sync_copy(x_hbm.at[pl.ds(wid * CHUNK + k * ROWS, ROWS)], xbuf)
        pltpu.async_copy(xbuf, sx_hbm.at[dest_v.at[k]], sem).wait()


def _sc_scatter(x, e_chunks, lr, hist):
    f = pl.kernel(
        _scatter_body,
        out_type=(
            jax.ShapeDtypeStruct((N, HIDDEN), jnp.float32),
            jax.ShapeDtypeStruct((NW, SUB, ROWS), jnp.int32),
            jax.ShapeDtypeStruct((MROWS, MAX_STEPS), jnp.int32),
        ),
        mesh=_sc_mesh(),
        compiler_params=pltpu.CompilerParams(needs_layout_passes=False),
        scratch_types=[
            pltpu.VMEM((NW, E), jnp.int32),
            pltpu.VMEM((E,), jnp.int32),
            pltpu.VMEM((CHUNK,), jnp.int32),
            pltpu.VMEM((CHUNK,), jnp.int32),
            pltpu.VMEM((SUB, ROWS), jnp.int32),
            pltpu.VMEM((ROWS, HIDDEN), jnp.float32),
            pltpu.VMEM((E,), jnp.int32),
            pltpu.VMEM((E,), jnp.int32),
            pltpu.VMEM((MAX_STEPS,), jnp.int32),
            pltpu.VMEM((MAX_STEPS,), jnp.int32),
            pltpu.VMEM((MROWS, MAX_STEPS), jnp.int32),
            pltpu.SemaphoreType.DMA,
        ],
    )
    return f(x, e_chunks, lr, hist)


def _unsort_body(os_hbm, dest_hbm, fin_hbm, dest_v, buf, sem):
    wid = _wid()
    pltpu.sync_copy(dest_hbm.at[wid], dest_v)
    for k in range(SUB):
        pltpu.async_copy(os_hbm.at[dest_v.at[k]], buf, sem).wait()
        pltpu.sync_copy(buf, fin_hbm.at[pl.ds(wid * CHUNK + k * ROWS, ROWS)])


def _sc_unsort(out_sorted, dest):
    f = pl.kernel(
        _unsort_body,
        out_type=jax.ShapeDtypeStruct((N, HIDDEN), jnp.float32),
        mesh=_sc_mesh(),
        compiler_params=pltpu.CompilerParams(needs_layout_passes=False),
        scratch_types=[
            pltpu.VMEM((SUB, ROWS), jnp.int32),
            pltpu.VMEM((ROWS, HIDDEN), jnp.float32),
            pltpu.SemaphoreType.DMA,
        ],
    )
    return f(out_sorted, dest)


def _gmm_body(meta_ref, x_hbm, gu_hbm, dn_hbm, o_hbm,
              xb, gub, dnb, ob, xs, gs, gs2, ds, osm):
    def issue(w):
        tile = meta_ref[0, w]
        e = meta_ref[1, w]
        xsl = meta_ref[6, w]
        wsl = meta_ref[7, w]

        @pl.when(meta_ref[4, w] == 1)
        def _():
            pltpu.make_async_copy(x_hbm.at[pl.ds(tile * T, T)],
                                  xb.at[xsl], xs.at[xsl]).start()

        @pl.when(meta_ref[5, w] == 1)
        def _():
            pltpu.make_async_copy(gu_hbm.at[e, pl.ds(0, HIDDEN // 2)],
                                  gub.at[wsl, pl.ds(0, HIDDEN // 2)],
                                  gs.at[wsl]).start()
            pltpu.make_async_copy(gu_hbm.at[e, pl.ds(HIDDEN // 2, HIDDEN // 2)],
                                  gub.at[wsl, pl.ds(HIDDEN // 2, HIDDEN // 2)],
                                  gs2.at[wsl]).start()
            pltpu.make_async_copy(dn_hbm.at[e], dnb.at[wsl], ds.at[wsl]).start()

    issue(0)
    rows = lax.broadcasted_iota(jnp.int32, (T, 1), 0)

    def step(w, carry):
        tile = meta_ref[0, w]
        e = meta_ref[1, w]
        rs = meta_ref[2, w]
        re_ = meta_ref[3, w]
        first = meta_ref[4, w]
        wchg = meta_ref[5, w]
        xsl = meta_ref[6, w]
        wsl = meta_ref[7, w]

        @pl.when(w + 1 < MAX_STEPS)
        def _():
            issue(w + 1)

        @pl.when(first == 1)
        def _():
            pltpu.make_async_copy(x_hbm.at[pl.ds(tile * T, T)],
                                  xb.at[xsl], xs.at[xsl]).wait()

        @pl.when(wchg == 1)
        def _():
            pltpu.make_async_copy(gu_hbm.at[e, pl.ds(0, HIDDEN // 2)],
                                  gub.at[wsl, pl.ds(0, HIDDEN // 2)],
                                  gs.at[wsl]).wait()
            pltpu.make_async_copy(gu_hbm.at[e, pl.ds(HIDDEN // 2, HIDDEN // 2)],
                                  gub.at[wsl, pl.ds(HIDDEN // 2, HIDDEN // 2)],
                                  gs2.at[wsl]).wait()
            pltpu.make_async_copy(dn_hbm.at[e], dnb.at[wsl], ds.at[wsl]).wait()

        @pl.when(meta_ref[8, w] == 1)
        def _():
            pltpu.make_async_copy(ob.at[xsl], o_hbm.at[pl.ds(tile * T, T)],
                                  osm.at[xsl]).wait()

        @pl.when(re_ > rs)
        def _():
            x = xb[xsl]
            gu = jnp.dot(x, gub[wsl], preferred_element_type=jnp.float32)
            gate = gu[:, :EI]
            up = gu[:, EI:]
            inter = gate * jax.nn.sigmoid(gate) * up
            part = jnp.dot(inter, dnb[wsl], preferred_element_type=jnp.float32)
            mask = (rows >= rs) & (rows < re_)

            @pl.when(first == 1)
            def _():
                ob[xsl] = jnp.where(mask, part, 0.0)

            @pl.when(first == 0)
            def _():
                ob[xsl] = jnp.where(mask, part, ob[xsl])

        @pl.when(meta_ref[9, w] == 1)
        def _():
            pltpu.make_async_copy(ob.at[xsl], o_hbm.at[pl.ds(tile * T, T)],
                                  osm.at[xsl]).start()
        return carry

    lax.fori_loop(0, MAX_STEPS, step, 0)
    pltpu.make_async_copy(ob.at[0], o_hbm.at[pl.ds(0, T)], osm.at[0]).wait()
    pltpu.make_async_copy(ob.at[1], o_hbm.at[pl.ds(0, T)], osm.at[1]).wait()


def _grouped_mlp(sorted_x, gate_up_proj, down_proj, meta):
    return pl.pallas_call(
        _gmm_body,
        in_specs=[
            pl.BlockSpec(memory_space=pltpu.SMEM),
            pl.BlockSpec(memory_space=pl.ANY),
            pl.BlockSpec(memory_space=pl.ANY),
            pl.BlockSpec(memory_space=pl.ANY),
        ],
        out_specs=pl.BlockSpec(memory_space=pl.ANY),
        out_shape=jax.ShapeDtypeStruct((N, HIDDEN), jnp.float32),
        scratch_shapes=[
            pltpu.VMEM((2, T, HIDDEN), jnp.float32),
            pltpu.VMEM((2, HIDDEN, 2 * EI), jnp.float32),
            pltpu.VMEM((2, EI, HIDDEN), jnp.float32),
            pltpu.VMEM((2, T, HIDDEN), jnp.float32),
            pltpu.SemaphoreType.DMA((2,)),
            pltpu.SemaphoreType.DMA((2,)),
            pltpu.SemaphoreType.DMA((2,)),
            pltpu.SemaphoreType.DMA((2,)),
            pltpu.SemaphoreType.DMA((2,)),
        ],
    )(meta, sorted_x, gate_up_proj, down_proj)


def kernel(x, token_ids, gate_up_proj, down_proj, token_to_expert):
    e_chunks, lr, hist = _sc_route(token_ids.astype(jnp.int32),
                                   token_to_expert.astype(jnp.int32))
    sorted_x, dest, meta = _sc_scatter(x, e_chunks, lr, hist)
    out_sorted = _grouped_mlp(sorted_x, gate_up_proj, down_proj, meta)
    return _sc_unsort(out_sorted, dest)


# speedup vs baseline: 1.2718x; 1.2718x over previous
import jax
import jax.numpy as jnp
from jax import lax
from jax.experimental import pallas as pl
from jax.experimental.pallas import tpu as pltpu
from jax.experimental.pallas import tpu_sc as plsc

HIDDEN = 1024
INTERMEDIATE = 8192
E = 64
VOCAB = 100000
EI = INTERMEDIATE // E
N = 8192

T = 128
NT = N // T
MAX_STEPS = NT + E

NC = 2
NS = 16
NW = NC * NS
L = 16
CHUNK = N // NW
VREGS = CHUNK // L
ROWS = 64
SUB = CHUNK // ROWS
MROWS = 10
NBUF = 3

def _sc_mesh():
    return plsc.VectorSubcoreMesh(core_axis_name="c", subcore_axis_name="s",
                                  num_cores=NC, num_subcores=NS)


def _wid():
    return lax.axis_index("s") * NC + lax.axis_index("c")


def _run_ranks(sk, scratch_ref):
    lane = lax.iota(jnp.int32, L)
    scratch_ref[...] = sk
    prev = plsc.load_gather(scratch_ref, [jnp.maximum(lane - 1, 0)])
    nxt = plsc.load_gather(scratch_ref, [jnp.minimum(lane + 1, L - 1)])
    is_start = (lane == 0) | (sk != prev)
    is_last = (lane == L - 1) | (sk != nxt)
    run_start = plsc.cummax(jnp.where(is_start, lane, 0))
    return lane - run_start, is_last


def _route_body(tids_hbm, tte_hbm, e_hbm, lr_hbm, hist_hbm,
                ids_v, tbl_v, e_v, lr_v, hist_v, tmp_v):
    wid = _wid()
    pltpu.sync_copy(tids_hbm.at[pl.ds(wid * CHUNK, CHUNK)], ids_v)
    pltpu.sync_copy(tte_hbm, tbl_v)
    lane = lax.iota(jnp.int32, L)
    for g in range(E // L):
        hist_v[pl.ds(g * L, L)] = jnp.zeros((L,), jnp.int32)
    for v in range(VREGS):
        idv = ids_v[pl.ds(v * L, L)]
        idv = jnp.minimum(jnp.maximum(idv, 0), VOCAB - 1)
        e16 = plsc.load_gather(tbl_v, [idv])
        e_v[pl.ds(v * L, L)] = e16
        sk, sv = plsc.sort_key_val(e16, lane)
        r, is_last = _run_ranks(sk, tmp_v)
        base = plsc.load_gather(hist_v, [sk])
        plsc.store_scatter(tmp_v, [sv], base + r)
        lr_v[pl.ds(v * L, L)] = tmp_v[...]
        plsc.addupdate_scatter(hist_v, [sk], r + 1, mask=is_last)
    pltpu.sync_copy(e_v, e_hbm.at[wid])
    pltpu.sync_copy(lr_v, lr_hbm.at[wid])
    pltpu.sync_copy(hist_v, hist_hbm.at[wid])


def _sc_route(token_ids, token_to_expert):
    f = pl.kernel(
        _route_body,
        out_type=(
            jax.ShapeDtypeStruct((NW, CHUNK), jnp.int32),
            jax.ShapeDtypeStruct((NW, CHUNK), jnp.int32),
            jax.ShapeDtypeStruct((NW, E), jnp.int32),
        ),
        mesh=_sc_mesh(),
        compiler_params=pltpu.CompilerParams(needs_layout_passes=False),
        scratch_types=[
            pltpu.VMEM((CHUNK,), jnp.int32),
            pltpu.VMEM((VOCAB,), jnp.int32),
            pltpu.VMEM((CHUNK,), jnp.int32),
            pltpu.VMEM((CHUNK,), jnp.int32),
            pltpu.VMEM((E,), jnp.int32),
            pltpu.VMEM((L,), jnp.int32),
        ],
    )
    return f(token_ids, token_to_expert)


_NEG = -(2 ** 30)


def _cummax_fill(buf_ref):
    carry = jnp.int32(_NEG)
    for q in range(MAX_STEPS // L):
        v = plsc.cummax(jnp.maximum(buf_ref[pl.ds(q * L, L)], carry))
        buf_ref[pl.ds(q * L, L)] = v
        carry = jnp.max(v)


def _build_meta(st_v, en_v, tot_by_g, total_tokens_unused, tb_v, eb_v, meta_v):
    lane = lax.iota(jnp.int32, L)
    carry = jnp.int32(0)
    for q in range(MAX_STEPS // L):
        tb_v[pl.ds(q * L, L)] = jnp.full((L,), _NEG, jnp.int32)
        eb_v[pl.ds(q * L, L)] = jnp.full((L,), _NEG, jnp.int32)
    for g in range(E // L):
        ev = jnp.int32(g * L) + lane
        starts = st_v[pl.ds(g * L, L)]
        ends = en_v[pl.ds(g * L, L)]
        tot = tot_by_g[g]
        ft = starts // T
        lt = jnp.maximum(ends - 1, 0) // T
        nsteps = jnp.where(tot > 0, lt - ft + 1, 0)
        inc = plsc.cumsum(nsteps) + carry
        off = inc - nsteps
        carry = jnp.max(inc)
        nz = tot > 0
        plsc.store_scatter(tb_v, [jnp.where(nz, off, 0)], off - ft, mask=nz)
        plsc.store_scatter(eb_v, [jnp.where(nz, off, 0)], ev, mask=nz)
    total = carry
    _cummax_fill(tb_v)
    _cummax_fill(eb_v)
    for q in range(MAX_STEPS // L):
        w = jnp.int32(q * L) + lane
        tile = jnp.minimum(w - tb_v[pl.ds(q * L, L)], NT - 1)
        e_w = eb_v[pl.ds(q * L, L)]
        starts = plsc.load_gather(st_v, [e_w])
        ends = plsc.load_gather(en_v, [e_w])
        valid = w < total
        rs = jnp.where(valid, jnp.maximum(starts - tile * T, 0), 0)
        re_ = jnp.where(valid, jnp.minimum(ends - tile * T, T), 0)
        meta_v[0, pl.ds(q * L, L)] = tile
        meta_v[1, pl.ds(q * L, L)] = e_w
        meta_v[2, pl.ds(q * L, L)] = rs
        meta_v[3, pl.ds(q * L, L)] = re_
    for q in range(MAX_STEPS // L):
        w = jnp.int32(q * L) + lane
        tile = meta_v[0, pl.ds(q * L, L)]
        prev = plsc.load_gather(tb_v, [jnp.maximum(w - 1, 0)])
        prev = jnp.minimum(jnp.maximum(w - 1, 0) - prev, NT - 1)
        meta_v[4, pl.ds(q * L, L)] = ((w == 0) | (tile != prev)).astype(jnp.int32)
    fcarry = jnp.int32(0)
    wcarry = jnp.int32(0)
    one_v = jnp.full((L,), 1, jnp.int32)
    for q in range(MAX_STEPS // L):
        w = jnp.int32(q * L) + lane
        e_w = meta_v[1, pl.ds(q * L, L)]
        prev_e = plsc.load_gather(eb_v, [jnp.maximum(w - 1, 0)])
        wchg = ((w == 0) | (e_w != prev_e)).astype(jnp.int32)
        meta_v[5, pl.ds(q * L, L)] = wchg
        first = meta_v[4, pl.ds(q * L, L)]
        visits = plsc.cumsum(first) + fcarry
        fcarry = jnp.max(visits)
        wcnt = plsc.cumsum(wchg) + wcarry
        wcarry = jnp.max(wcnt)
        meta_v[6, pl.ds(q * L, L)] = (visits - 1) % NBUF
        meta_v[7, pl.ds(q * L, L)] = (wcnt - 1) % NBUF
        meta_v[8, pl.ds(q * L, L)] = first * (visits > NBUF).astype(jnp.int32)
    for q in range(MAX_STEPS // L):
        w = jnp.int32(q * L) + lane
        nxt_first = plsc.load_gather(
            meta_v, [jnp.full((L,), 4, jnp.int32),
                     jnp.minimum(w + 1, MAX_STEPS - 1)])
        meta_v[9, pl.ds(q * L, L)] = jnp.where(
            w == MAX_STEPS - 1, 1, nxt_first)


def _scatter_body(x_hbm, e_hbm, lr_hbm, hist_hbm, sx_hbm, dest_hbm, meta_hbm,
                  hv, base_v, e_v, lr_v, dest_v, xbuf, st_v, en_v, tb_v, eb_v,
                  meta_v, sem):
    wid = _wid()
    pltpu.sync_copy(hist_hbm, hv)
    pltpu.sync_copy(e_hbm.at[wid], e_v)
    pltpu.sync_copy(lr_hbm.at[wid], lr_v)
    carry = jnp.int32(0)
    tot_by_g = []
    for g in range(E // L):
        tot = jnp.zeros((L,), jnp.int32)
        mine = jnp.zeros((L,), jnp.int32)
        for t in range(NW):
            h = hv[t, pl.ds(g * L, L)]
            tot = tot + h
            mine = mine + h * (jnp.int32(t) < wid).astype(jnp.int32)
        excl = plsc.cumsum(tot) - tot
        starts = excl + carry
        st_v[pl.ds(g * L, L)] = starts
        en_v[pl.ds(g * L, L)] = starts + tot
        base_v[pl.ds(g * L, L)] = starts + mine
        carry = carry + jnp.sum(tot)
        tot_by_g.append(tot)

    @pl.when(wid == 0)
    def _():
        _build_meta(st_v, en_v, tot_by_g, carry, tb_v, eb_v, meta_v)
        pltpu.sync_copy(meta_v, meta_hbm)

    for v in range(VREGS):
        e16 = e_v[pl.ds(v * L, L)]
        lr16 = lr_v[pl.ds(v * L, L)]
        d16 = plsc.load_gather(base_v, [e16]) + lr16
        dest_v[v // (ROWS // L), pl.ds((v % (ROWS // L)) * L, L)] = d16
    pltpu.sync_copy(dest_v, dest_hbm.at[wid])
    for k in range(SUB):
        pltpu.sync_copy(x_hbm.at[pl.ds(wid * CHUNK + k * ROWS, ROWS)], xbuf)
        pltpu.async_copy(xbuf, sx_hbm.at[dest_v.at[k]], sem).wait()


def _sc_scatter(x, e_chunks, lr, hist):
    f = pl.kernel(
        _scatter_body,
        out_type=(
            jax.ShapeDtypeStruct((N, HIDDEN), jnp.float32),
            jax.ShapeDtypeStruct((NW, SUB, ROWS), jnp.int32),
            jax.ShapeDtypeStruct((MROWS, MAX_STEPS), jnp.int32),
        ),
        mesh=_sc_mesh(),
        compiler_params=pltpu.CompilerParams(needs_layout_passes=False),
        scratch_types=[
            pltpu.VMEM((NW, E), jnp.int32),
            pltpu.VMEM((E,), jnp.int32),
            pltpu.VMEM((CHUNK,), jnp.int32),
            pltpu.VMEM((CHUNK,), jnp.int32),
            pltpu.VMEM((SUB, ROWS), jnp.int32),
            pltpu.VMEM((ROWS, HIDDEN), jnp.float32),
            pltpu.VMEM((E,), jnp.int32),
            pltpu.VMEM((E,), jnp.int32),
            pltpu.VMEM((MAX_STEPS,), jnp.int32),
            pltpu.VMEM((MAX_STEPS,), jnp.int32),
            pltpu.VMEM((MROWS, MAX_STEPS), jnp.int32),
            pltpu.SemaphoreType.DMA,
        ],
    )
    return f(x, e_chunks, lr, hist)


def _unsort_body(os_hbm, dest_hbm, fin_hbm, dest_v, buf, sem):
    wid = _wid()
    pltpu.sync_copy(dest_hbm.at[wid], dest_v)
    for k in range(SUB):
        pltpu.async_copy(os_hbm.at[dest_v.at[k]], buf, sem).wait()
        pltpu.sync_copy(buf, fin_hbm.at[pl.ds(wid * CHUNK + k * ROWS, ROWS)])


def _sc_unsort(out_sorted, dest):
    f = pl.kernel(
        _unsort_body,
        out_type=jax.ShapeDtypeStruct((N, HIDDEN), jnp.float32),
        mesh=_sc_mesh(),
        compiler_params=pltpu.CompilerParams(needs_layout_passes=False),
        scratch_types=[
            pltpu.VMEM((SUB, ROWS), jnp.int32),
            pltpu.VMEM((ROWS, HIDDEN), jnp.float32),
            pltpu.SemaphoreType.DMA,
        ],
    )
    return f(out_sorted, dest)


def _gmm_body(meta_ref, x_hbm, gu_hbm, dn_hbm, o_hbm,
              xb, gub, dnb, ob, xs, gs, gs2, ds, osm):
    def issue(w):
        tile = meta_ref[0, w]
        e = meta_ref[1, w]
        xsl = meta_ref[6, w]
        wsl = meta_ref[7, w]

        @pl.when(meta_ref[4, w] == 1)
        def _():
            pltpu.make_async_copy(x_hbm.at[pl.ds(tile * T, T)],
                                  xb.at[xsl], xs.at[xsl]).start()

        @pl.when(meta_ref[5, w] == 1)
        def _():
            pltpu.make_async_copy(gu_hbm.at[e, pl.ds(0, HIDDEN // 2)],
                                  gub.at[wsl, pl.ds(0, HIDDEN // 2)],
                                  gs.at[wsl]).start()
            pltpu.make_async_copy(gu_hbm.at[e, pl.ds(HIDDEN // 2, HIDDEN // 2)],
                                  gub.at[wsl, pl.ds(HIDDEN // 2, HIDDEN // 2)],
                                  gs2.at[wsl]).start()
            pltpu.make_async_copy(dn_hbm.at[e], dnb.at[wsl], ds.at[wsl]).start()

    issue(0)
    issue(1)
    rows = lax.broadcasted_iota(jnp.int32, (T, 1), 0)

    def step(w, carry):
        tile = meta_ref[0, w]
        e = meta_ref[1, w]
        rs = meta_ref[2, w]
        re_ = meta_ref[3, w]
        first = meta_ref[4, w]
        wchg = meta_ref[5, w]
        xsl = meta_ref[6, w]
        wsl = meta_ref[7, w]

        @pl.when(w + 2 < MAX_STEPS)
        def _():
            issue(w + 2)

        @pl.when(first == 1)
        def _():
            pltpu.make_async_copy(x_hbm.at[pl.ds(tile * T, T)],
                                  xb.at[xsl], xs.at[xsl]).wait()

        @pl.when(wchg == 1)
        def _():
            pltpu.make_async_copy(gu_hbm.at[e, pl.ds(0, HIDDEN // 2)],
                                  gub.at[wsl, pl.ds(0, HIDDEN // 2)],
                                  gs.at[wsl]).wait()
            pltpu.make_async_copy(gu_hbm.at[e, pl.ds(HIDDEN // 2, HIDDEN // 2)],
                                  gub.at[wsl, pl.ds(HIDDEN // 2, HIDDEN // 2)],
                                  gs2.at[wsl]).wait()
            pltpu.make_async_copy(dn_hbm.at[e], dnb.at[wsl], ds.at[wsl]).wait()

        @pl.when(meta_ref[8, w] == 1)
        def _():
            pltpu.make_async_copy(ob.at[xsl], o_hbm.at[pl.ds(tile * T, T)],
                                  osm.at[xsl]).wait()

        @pl.when(re_ > rs)
        def _():
            x = xb[xsl]
            gu = jnp.dot(x, gub[wsl], preferred_element_type=jnp.float32)
            gate = gu[:, :EI]
            up = gu[:, EI:]
            inter = gate * jax.nn.sigmoid(gate) * up
            part = jnp.dot(inter, dnb[wsl], preferred_element_type=jnp.float32)
            mask = (rows >= rs) & (rows < re_)

            @pl.when(first == 1)
            def _():
                ob[xsl] = jnp.where(mask, part, 0.0)

            @pl.when(first == 0)
            def _():
                ob[xsl] = jnp.where(mask, part, ob[xsl])

        @pl.when(meta_ref[9, w] == 1)
        def _():
            pltpu.make_async_copy(ob.at[xsl], o_hbm.at[pl.ds(tile * T, T)],
                                  osm.at[xsl]).start()
        return carry

    lax.fori_loop(0, MAX_STEPS, step, 0)
    for s in range(NBUF):
        pltpu.make_async_copy(ob.at[s], o_hbm.at[pl.ds(0, T)], osm.at[s]).wait()


def _grouped_mlp(sorted_x, gate_up_proj, down_proj, meta):
    return pl.pallas_call(
        _gmm_body,
        in_specs=[
            pl.BlockSpec(memory_space=pltpu.SMEM),
            pl.BlockSpec(memory_space=pl.ANY),
            pl.BlockSpec(memory_space=pl.ANY),
            pl.BlockSpec(memory_space=pl.ANY),
        ],
        out_specs=pl.BlockSpec(memory_space=pl.ANY),
        out_shape=jax.ShapeDtypeStruct((N, HIDDEN), jnp.float32),
        scratch_shapes=[
            pltpu.VMEM((NBUF, T, HIDDEN), jnp.float32),
            pltpu.VMEM((NBUF, HIDDEN, 2 * EI), jnp.float32),
            pltpu.VMEM((NBUF, EI, HIDDEN), jnp.float32),
            pltpu.VMEM((NBUF, T, HIDDEN), jnp.float32),
            pltpu.SemaphoreType.DMA((NBUF,)),
            pltpu.SemaphoreType.DMA((NBUF,)),
            pltpu.SemaphoreType.DMA((NBUF,)),
            pltpu.SemaphoreType.DMA((NBUF,)),
            pltpu.SemaphoreType.DMA((NBUF,)),
        ],
    )(meta, sorted_x, gate_up_proj, down_proj)


def kernel(x, token_ids, gate_up_proj, down_proj, token_to_expert):
    e_chunks, lr, hist = _sc_route(token_ids.astype(jnp.int32),
                                   token_to_expert.astype(jnp.int32))
    sorted_x, dest, meta = _sc_scatter(x, e_chunks, lr, hist)
    out_sorted = _grouped_mlp(sorted_x, gate_up_proj, down_proj, meta)
    return _sc_unsort(out_sorted, dest)


# speedup vs baseline: 1.3666x; 1.0746x over previous
import jax
import jax.numpy as jnp
from jax import lax
from jax.experimental import pallas as pl
from jax.experimental.pallas import tpu as pltpu
from jax.experimental.pallas import tpu_sc as plsc

HIDDEN = 1024
INTERMEDIATE = 8192
E = 64
VOCAB = 100000
EI = INTERMEDIATE // E
N = 8192

T = 128
NT = N // T
MAX_STEPS = NT + E

NC = 2
NS = 16
NW = NC * NS
L = 16
CHUNK = N // NW
VREGS = CHUNK // L
ROWS = 64
SUB = CHUNK // ROWS
MROWS = 10
NBUF = 4
PREF = NBUF - 1

def _sc_mesh():
    return plsc.VectorSubcoreMesh(core_axis_name="c", subcore_axis_name="s",
                                  num_cores=NC, num_subcores=NS)


def _wid():
    return lax.axis_index("s") * NC + lax.axis_index("c")


def _run_ranks(sk, scratch_ref):
    lane = lax.iota(jnp.int32, L)
    scratch_ref[...] = sk
    prev = plsc.load_gather(scratch_ref, [jnp.maximum(lane - 1, 0)])
    nxt = plsc.load_gather(scratch_ref, [jnp.minimum(lane + 1, L - 1)])
    is_start = (lane == 0) | (sk != prev)
    is_last = (lane == L - 1) | (sk != nxt)
    run_start = plsc.cummax(jnp.where(is_start, lane, 0))
    return lane - run_start, is_last


def _route_body(tids_hbm, tte_hbm, e_hbm, lr_hbm, hist_hbm,
                ids_v, tbl_v, e_v, lr_v, hist_v, tmp_v):
    wid = _wid()
    pltpu.sync_copy(tids_hbm.at[pl.ds(wid * CHUNK, CHUNK)], ids_v)
    pltpu.sync_copy(tte_hbm, tbl_v)
    lane = lax.iota(jnp.int32, L)
    for g in range(E // L):
        hist_v[pl.ds(g * L, L)] = jnp.zeros((L,), jnp.int32)
    for v in range(VREGS):
        idv = ids_v[pl.ds(v * L, L)]
        idv = jnp.minimum(jnp.maximum(idv, 0), VOCAB - 1)
        e16 = plsc.load_gather(tbl_v, [idv])
        e_v[pl.ds(v * L, L)] = e16
        sk, sv = plsc.sort_key_val(e16, lane)
        r, is_last = _run_ranks(sk, tmp_v)
        base = plsc.load_gather(hist_v, [sk])
        plsc.store_scatter(tmp_v, [sv], base + r)
        lr_v[pl.ds(v * L, L)] = tmp_v[...]
        plsc.addupdate_scatter(hist_v, [sk], r + 1, mask=is_last)
    pltpu.sync_copy(e_v, e_hbm.at[wid])
    pltpu.sync_copy(lr_v, lr_hbm.at[wid])
    pltpu.sync_copy(hist_v, hist_hbm.at[wid])


def _sc_route(token_ids, token_to_expert):
    f = pl.kernel(
        _route_body,
        out_type=(
            jax.ShapeDtypeStruct((NW, CHUNK), jnp.int32),
            jax.ShapeDtypeStruct((NW, CHUNK), jnp.int32),
            jax.ShapeDtypeStruct((NW, E), jnp.int32),
        ),
        mesh=_sc_mesh(),
        compiler_params=pltpu.CompilerParams(needs_layout_passes=False),
        scratch_types=[
            pltpu.VMEM((CHUNK,), jnp.int32),
            pltpu.VMEM((VOCAB,), jnp.int32),
            pltpu.VMEM((CHUNK,), jnp.int32),
            pltpu.VMEM((CHUNK,), jnp.int32),
            pltpu.VMEM((E,), jnp.int32),
            pltpu.VMEM((L,), jnp.int32),
        ],
    )
    return f(token_ids, token_to_expert)


_NEG = -(2 ** 30)


def _cummax_fill(buf_ref):
    carry = jnp.int32(_NEG)
    for q in range(MAX_STEPS // L):
        v = plsc.cummax(jnp.maximum(buf_ref[pl.ds(q * L, L)], carry))
        buf_ref[pl.ds(q * L, L)] = v
        carry = jnp.max(v)


def _build_meta(st_v, en_v, tot_by_g, total_tokens_unused, tb_v, eb_v, meta_v):
    lane = lax.iota(jnp.int32, L)
    carry = jnp.int32(0)
    for q in range(MAX_STEPS // L):
        tb_v[pl.ds(q * L, L)] = jnp.full((L,), _NEG, jnp.int32)
        eb_v[pl.ds(q * L, L)] = jnp.full((L,), _NEG, jnp.int32)
    for g in range(E // L):
        ev = jnp.int32(g * L) + lane
        starts = st_v[pl.ds(g * L, L)]
        ends = en_v[pl.ds(g * L, L)]
        tot = tot_by_g[g]
        ft = starts // T
        lt = jnp.maximum(ends - 1, 0) // T
        nsteps = jnp.where(tot > 0, lt - ft + 1, 0)
        inc = plsc.cumsum(nsteps) + carry
        off = inc - nsteps
        carry = jnp.max(inc)
        nz = tot > 0
        plsc.store_scatter(tb_v, [jnp.where(nz, off, 0)], off - ft, mask=nz)
        plsc.store_scatter(eb_v, [jnp.where(nz, off, 0)], ev, mask=nz)
    total = carry
    _cummax_fill(tb_v)
    _cummax_fill(eb_v)
    for q in range(MAX_STEPS // L):
        w = jnp.int32(q * L) + lane
        tile = jnp.minimum(w - tb_v[pl.ds(q * L, L)], NT - 1)
        e_w = eb_v[pl.ds(q * L, L)]
        starts = plsc.load_gather(st_v, [e_w])
        ends = plsc.load_gather(en_v, [e_w])
        valid = w < total
        rs = jnp.where(valid, jnp.maximum(starts - tile * T, 0), 0)
        re_ = jnp.where(valid, jnp.minimum(ends - tile * T, T), 0)
        meta_v[0, pl.ds(q * L, L)] = tile
        meta_v[1, pl.ds(q * L, L)] = e_w
        meta_v[2, pl.ds(q * L, L)] = rs
        meta_v[3, pl.ds(q * L, L)] = re_
    for q in range(MAX_STEPS // L):
        w = jnp.int32(q * L) + lane
        tile = meta_v[0, pl.ds(q * L, L)]
        prev = plsc.load_gather(tb_v, [jnp.maximum(w - 1, 0)])
        prev = jnp.minimum(jnp.maximum(w - 1, 0) - prev, NT - 1)
        meta_v[4, pl.ds(q * L, L)] = ((w == 0) | (tile != prev)).astype(jnp.int32)
    fcarry = jnp.int32(0)
    wcarry = jnp.int32(0)
    one_v = jnp.full((L,), 1, jnp.int32)
    for q in range(MAX_STEPS // L):
        w = jnp.int32(q * L) + lane
        e_w = meta_v[1, pl.ds(q * L, L)]
        prev_e = plsc.load_gather(eb_v, [jnp.maximum(w - 1, 0)])
        wchg = ((w == 0) | (e_w != prev_e)).astype(jnp.int32)
        meta_v[5, pl.ds(q * L, L)] = wchg
        first = meta_v[4, pl.ds(q * L, L)]
        visits = plsc.cumsum(first) + fcarry
        fcarry = jnp.max(visits)
        wcnt = plsc.cumsum(wchg) + wcarry
        wcarry = jnp.max(wcnt)
        meta_v[6, pl.ds(q * L, L)] = (visits - 1) % NBUF
        meta_v[7, pl.ds(q * L, L)] = (wcnt - 1) % NBUF
        meta_v[8, pl.ds(q * L, L)] = first * (visits > NBUF).astype(jnp.int32)
    for q in range(MAX_STEPS // L):
        w = jnp.int32(q * L) + lane
        nxt_first = plsc.load_gather(
            meta_v, [jnp.full((L,), 4, jnp.int32),
                     jnp.minimum(w + 1, MAX_STEPS - 1)])
        meta_v[9, pl.ds(q * L, L)] = jnp.where(
            w == MAX_STEPS - 1, 1, nxt_first)


def _scatter_body(x_hbm, e_hbm, lr_hbm, hist_hbm, sx_hbm, dest_hbm, meta_hbm,
                  hv, base_v, e_v, lr_v, dest_v, xbuf, st_v, en_v, tb_v, eb_v,
                  meta_v, sem):
    wid = _wid()
    pltpu.sync_copy(hist_hbm, hv)
    pltpu.sync_copy(e_hbm.at[wid], e_v)
    pltpu.sync_copy(lr_hbm.at[wid], lr_v)
    carry = jnp.int32(0)
    tot_by_g = []
    for g in range(E // L):
        tot = jnp.zeros((L,), jnp.int32)
        mine = jnp.zeros((L,), jnp.int32)
        for t in range(NW):
            h = hv[t, pl.ds(g * L, L)]
            tot = tot + h
            mine = mine + h * (jnp.int32(t) < wid).astype(jnp.int32)
        excl = plsc.cumsum(tot) - tot
        starts = excl + carry
        st_v[pl.ds(g * L, L)] = starts
        en_v[pl.ds(g * L, L)] = starts + tot
        base_v[pl.ds(g * L, L)] = starts + mine
        carry = carry + jnp.sum(tot)
        tot_by_g.append(tot)

    @pl.when(wid == 0)
    def _():
        _build_meta(st_v, en_v, tot_by_g, carry, tb_v, eb_v, meta_v)
        pltpu.sync_copy(meta_v, meta_hbm)

    for v in range(VREGS):
        e16 = e_v[pl.ds(v * L, L)]
        lr16 = lr_v[pl.ds(v * L, L)]
        d16 = plsc.load_gather(base_v, [e16]) + lr16
        dest_v[v // (ROWS // L), pl.ds((v % (ROWS // L)) * L, L)] = d16
    pltpu.sync_copy(dest_v, dest_hbm.at[wid])
    for k in range(SUB):
        pltpu.sync_copy(x_hbm.at[pl.ds(wid * CHUNK + k * ROWS, ROWS)], xbuf)
        pltpu.async_copy(xbuf, sx_hbm.at[dest_v.at[k]], sem).wait()


def _sc_scatter(x, e_chunks, lr, hist):
    f = pl.kernel(
        _scatter_body,
        out_type=(
            jax.ShapeDtypeStruct((N, HIDDEN), jnp.float32),
            jax.ShapeDtypeStruct((NW, SUB, ROWS), jnp.int32),
            jax.ShapeDtypeStruct((MROWS, MAX_STEPS), jnp.int32),
        ),
        mesh=_sc_mesh(),
        compiler_params=pltpu.CompilerParams(needs_layout_passes=False),
        scratch_types=[
            pltpu.VMEM((NW, E), jnp.int32),
            pltpu.VMEM((E,), jnp.int32),
            pltpu.VMEM((CHUNK,), jnp.int32),
            pltpu.VMEM((CHUNK,), jnp.int32),
            pltpu.VMEM((SUB, ROWS), jnp.int32),
            pltpu.VMEM((ROWS, HIDDEN), jnp.float32),
            pltpu.VMEM((E,), jnp.int32),
            pltpu.VMEM((E,), jnp.int32),
            pltpu.VMEM((MAX_STEPS,), jnp.int32),
            pltpu.VMEM((MAX_STEPS,), jnp.int32),
            pltpu.VMEM((MROWS, MAX_STEPS), jnp.int32),
            pltpu.SemaphoreType.DMA,
        ],
    )
    return f(x, e_chunks, lr, hist)


def _unsort_body(os_hbm, dest_hbm, fin_hbm, dest_v, buf, sem):
    wid = _wid()
    pltpu.sync_copy(dest_hbm.at[wid], dest_v)
    for k in range(SUB):
        pltpu.async_copy(os_hbm.at[dest_v.at[k]], buf, sem).wait()
        pltpu.sync_copy(buf, fin_hbm.at[pl.ds(wid * CHUNK + k * ROWS, ROWS)])


def _sc_unsort(out_sorted, dest):
    f = pl.kernel(
        _unsort_body,
        out_type=jax.ShapeDtypeStruct((N, HIDDEN), jnp.float32),
        mesh=_sc_mesh(),
        compiler_params=pltpu.CompilerParams(needs_layout_passes=False),
        scratch_types=[
            pltpu.VMEM((SUB, ROWS), jnp.int32),
            pltpu.VMEM((ROWS, HIDDEN), jnp.float32),
            pltpu.SemaphoreType.DMA,
        ],
    )
    return f(out_sorted, dest)


def _gmm_body(meta_ref, x_hbm, gu_hbm, dn_hbm, o_hbm,
              xb, gub, dnb, ob, xs, gs, gs2, ds, osm):
    def issue(w):
        tile = meta_ref[0, w]
        e = meta_ref[1, w]
        xsl = meta_ref[6, w]
        wsl = meta_ref[7, w]

        @pl.when(meta_ref[4, w] == 1)
        def _():
            pltpu.make_async_copy(x_hbm.at[pl.ds(tile * T, T)],
                                  xb.at[xsl], xs.at[xsl]).start()

        @pl.when(meta_ref[5, w] == 1)
        def _():
            pltpu.make_async_copy(gu_hbm.at[e, pl.ds(0, HIDDEN // 2)],
                                  gub.at[wsl, pl.ds(0, HIDDEN // 2)],
                                  gs.at[wsl]).start()
            pltpu.make_async_copy(gu_hbm.at[e, pl.ds(HIDDEN // 2, HIDDEN // 2)],
                                  gub.at[wsl, pl.ds(HIDDEN // 2, HIDDEN // 2)],
                                  gs2.at[wsl]).start()
            pltpu.make_async_copy(dn_hbm.at[e], dnb.at[wsl], ds.at[wsl]).start()

    for p in range(PREF):
        issue(p)
    rows = lax.broadcasted_iota(jnp.int32, (T, 1), 0)

    def step(w, carry):
        tile = meta_ref[0, w]
        e = meta_ref[1, w]
        rs = meta_ref[2, w]
        re_ = meta_ref[3, w]
        first = meta_ref[4, w]
        wchg = meta_ref[5, w]
        xsl = meta_ref[6, w]
        wsl = meta_ref[7, w]

        @pl.when(w + PREF < MAX_STEPS)
        def _():
            issue(w + PREF)

        @pl.when(first == 1)
        def _():
            pltpu.make_async_copy(x_hbm.at[pl.ds(tile * T, T)],
                                  xb.at[xsl], xs.at[xsl]).wait()

        @pl.when(wchg == 1)
        def _():
            pltpu.make_async_copy(gu_hbm.at[e, pl.ds(0, HIDDEN // 2)],
                                  gub.at[wsl, pl.ds(0, HIDDEN // 2)],
                                  gs.at[wsl]).wait()
            pltpu.make_async_copy(gu_hbm.at[e, pl.ds(HIDDEN // 2, HIDDEN // 2)],
                                  gub.at[wsl, pl.ds(HIDDEN // 2, HIDDEN // 2)],
                                  gs2.at[wsl]).wait()
            pltpu.make_async_copy(dn_hbm.at[e], dnb.at[wsl], ds.at[wsl]).wait()

        @pl.when(meta_ref[8, w] == 1)
        def _():
            pltpu.make_async_copy(ob.at[xsl], o_hbm.at[pl.ds(tile * T, T)],
                                  osm.at[xsl]).wait()

        @pl.when(re_ > rs)
        def _():
            x = xb[xsl]
            gu = jnp.dot(x, gub[wsl], preferred_element_type=jnp.float32)
            gate = gu[:, :EI]
            up = gu[:, EI:]
            inter = gate * jax.nn.sigmoid(gate) * up
            part = jnp.dot(inter, dnb[wsl], preferred_element_type=jnp.float32)
            mask = (rows >= rs) & (rows < re_)

            @pl.when(first == 1)
            def _():
                ob[xsl] = jnp.where(mask, part, 0.0)

            @pl.when(first == 0)
            def _():
                ob[xsl] = jnp.where(mask, part, ob[xsl])

        @pl.when(meta_ref[9, w] == 1)
        def _():
            pltpu.make_async_copy(ob.at[xsl], o_hbm.at[pl.ds(tile * T, T)],
                                  osm.at[xsl]).start()
        return carry

    lax.fori_loop(0, MAX_STEPS, step, 0)
    for s in range(NBUF):
        pltpu.make_async_copy(ob.at[s], o_hbm.at[pl.ds(0, T)], osm.at[s]).wait()


def _grouped_mlp(sorted_x, gate_up_proj, down_proj, meta):
    return pl.pallas_call(
        _gmm_body,
        in_specs=[
            pl.BlockSpec(memory_space=pltpu.SMEM),
            pl.BlockSpec(memory_space=pl.ANY),
            pl.BlockSpec(memory_space=pl.ANY),
            pl.BlockSpec(memory_space=pl.ANY),
        ],
        out_specs=pl.BlockSpec(memory_space=pl.ANY),
        out_shape=jax.ShapeDtypeStruct((N, HIDDEN), jnp.float32),
        scratch_shapes=[
            pltpu.VMEM((NBUF, T, HIDDEN), jnp.float32),
            pltpu.VMEM((NBUF, HIDDEN, 2 * EI), jnp.float32),
            pltpu.VMEM((NBUF, EI, HIDDEN), jnp.float32),
            pltpu.VMEM((NBUF, T, HIDDEN), jnp.float32),
            pltpu.SemaphoreType.DMA((NBUF,)),
            pltpu.SemaphoreType.DMA((NBUF,)),
            pltpu.SemaphoreType.DMA((NBUF,)),
            pltpu.SemaphoreType.DMA((NBUF,)),
            pltpu.SemaphoreType.DMA((NBUF,)),
        ],
    )(meta, sorted_x, gate_up_proj, down_proj)


def kernel(x, token_ids, gate_up_proj, down_proj, token_to_expert):
    e_chunks, lr, hist = _sc_route(token_ids.astype(jnp.int32),
                                   token_to_expert.astype(jnp.int32))
    sorted_x, dest, meta = _sc_scatter(x, e_chunks, lr, hist)
    out_sorted = _grouped_mlp(sorted_x, gate_up_proj, down_proj, meta)
    return _sc_unsort(out_sorted, dest)
